# baseline (speedup 1.0000x reference)
import functools

import jax
import jax.numpy as jnp
from jax import lax
from jax.experimental import pallas as pl
from jax.experimental.pallas import tpu as pltpu
from jax.experimental.pallas import tpu_sc as plsc


@functools.cache
def _make_copy_kernel(rows: int, dim: int):
    info = plsc.get_sparse_core_info()
    nw = info.num_cores * info.num_subcores
    rows_per_w = rows // nw
    mesh = plsc.VectorSubcoreMesh(core_axis_name="c", subcore_axis_name="s")

    @functools.partial(
        pl.kernel,
        mesh=mesh,
        out_type=jax.ShapeDtypeStruct((rows, dim), jnp.float32),
    )
    def k(emb_hbm, out_hbm):
        wid = lax.axis_index("s") * info.num_cores + lax.axis_index("c")
        base = wid * rows_per_w
        pltpu.sync_copy(
            emb_hbm.at[pl.ds(base, rows_per_w)],
            out_hbm.at[pl.ds(base, rows_per_w)],
        )

    return k


def kernel(x, emb_weight):
    rows = x.shape[1]
    return _make_copy_kernel(rows, emb_weight.shape[1])(emb_weight)


# speedup vs baseline: 24.2509x; 24.2509x over previous
import functools

import jax
import jax.numpy as jnp
from jax import lax
from jax.experimental import pallas as pl
from jax.experimental.pallas import tpu as pltpu
from jax.experimental.pallas import tpu_sc as plsc

_CHUNK = 16
_NBUF = 4


@functools.cache
def _make_copy_kernel(rows: int, dim: int):
    info = plsc.get_sparse_core_info()
    nc, ns = info.num_cores, info.num_subcores
    nw = nc * ns
    rows_per_w = rows // nw
    nchunks = rows_per_w // _CHUNK
    mesh = plsc.VectorSubcoreMesh(core_axis_name="c", subcore_axis_name="s")

    @functools.partial(
        pl.kernel,
        mesh=mesh,
        out_type=jax.ShapeDtypeStruct((rows, dim), jnp.float32),
        scratch_types=[
            pltpu.VMEM((_NBUF, _CHUNK, dim), jnp.float32),
            pltpu.SemaphoreType.DMA((_NBUF,)),
            pltpu.SemaphoreType.DMA((_NBUF,)),
        ],
    )
    def k(emb_hbm, out_hbm, bufs, gsem, ssem):
        wid = lax.axis_index("s") * nc + lax.axis_index("c")
        base = wid * rows_per_w

        def gather(i, b):
            return pltpu.async_copy(
                emb_hbm.at[pl.ds(base + i * _CHUNK, _CHUNK)],
                bufs.at[b],
                gsem.at[b],
            )

        def scatter(i, b):
            return pltpu.async_copy(
                bufs.at[b],
                out_hbm.at[pl.ds(base + i * _CHUNK, _CHUNK)],
                ssem.at[b],
            )

        g = [None] * nchunks
        s = [None] * nchunks
        for b in range(min(_NBUF, nchunks)):
            g[b] = gather(b, b)
        for i in range(nchunks):
            b = i % _NBUF
            g[i].wait()
            s[i] = scatter(i, b)
            nxt = i + _NBUF
            if nxt < nchunks:
                s[i].wait()
                g[nxt] = gather(nxt, b)
        for i in range(max(0, nchunks - _NBUF), nchunks):
            if s[i] is not None and i + _NBUF >= nchunks:
                s[i].wait()

    return k


def kernel(x, emb_weight):
    rows = x.shape[1]
    return _make_copy_kernel(rows, emb_weight.shape[1])(emb_weight)


# speedup vs baseline: 24.7646x; 1.0212x over previous
import functools

import jax
import jax.numpy as jnp
from jax import lax
from jax.experimental import pallas as pl
from jax.experimental.pallas import tpu as pltpu
from jax.experimental.pallas import tpu_sc as plsc

_CHUNK = 32
_NBUF = 3


@functools.cache
def _make_copy_kernel(rows: int, dim: int):
    info = plsc.get_sparse_core_info()
    nc, ns = info.num_cores, info.num_subcores
    nw = nc * ns
    rows_per_w = rows // nw
    nchunks = rows_per_w // _CHUNK
    mesh = plsc.VectorSubcoreMesh(core_axis_name="c", subcore_axis_name="s")

    @functools.partial(
        pl.kernel,
        mesh=mesh,
        out_type=jax.ShapeDtypeStruct((rows, dim), jnp.float32),
        scratch_types=[
            pltpu.VMEM((_NBUF, _CHUNK, dim), jnp.float32),
            pltpu.SemaphoreType.DMA((_NBUF,)),
            pltpu.SemaphoreType.DMA((_NBUF,)),
        ],
    )
    def k(emb_hbm, out_hbm, bufs, gsem, ssem):
        wid = lax.axis_index("s") * nc + lax.axis_index("c")
        base = wid * rows_per_w

        def gather(i, b):
            return pltpu.async_copy(
                emb_hbm.at[pl.ds(base + i * _CHUNK, _CHUNK)],
                bufs.at[b],
                gsem.at[b],
            )

        def scatter(i, b):
            return pltpu.async_copy(
                bufs.at[b],
                out_hbm.at[pl.ds(base + i * _CHUNK, _CHUNK)],
                ssem.at[b],
            )

        g = [None] * nchunks
        s = [None] * nchunks
        for b in range(min(_NBUF, nchunks)):
            g[b] = gather(b, b)
        for i in range(nchunks):
            b = i % _NBUF
            g[i].wait()
            s[i] = scatter(i, b)
            nxt = i + _NBUF
            if nxt < nchunks:
                s[i].wait()
                g[nxt] = gather(nxt, b)
        for i in range(max(0, nchunks - _NBUF), nchunks):
            if s[i] is not None and i + _NBUF >= nchunks:
                s[i].wait()

    return k


def kernel(x, emb_weight):
    rows = x.shape[1]
    return _make_copy_kernel(rows, emb_weight.shape[1])(emb_weight)


# speedup vs baseline: 24.9953x; 1.0093x over previous
import functools

import jax
import jax.numpy as jnp
from jax import lax
from jax.experimental import pallas as pl
from jax.experimental.pallas import tpu as pltpu
from jax.experimental.pallas import tpu_sc as plsc

_CHUNK = 16
_NBUF = 7


@functools.cache
def _make_copy_kernel(rows: int, dim: int):
    info = plsc.get_sparse_core_info()
    nc, ns = info.num_cores, info.num_subcores
    nw = nc * ns
    rows_per_w = rows // nw
    nchunks = rows_per_w // _CHUNK
    mesh = plsc.VectorSubcoreMesh(core_axis_name="c", subcore_axis_name="s")

    @functools.partial(
        pl.kernel,
        mesh=mesh,
        out_type=jax.ShapeDtypeStruct((rows, dim), jnp.float32),
        scratch_types=[
            pltpu.VMEM((_NBUF, _CHUNK, dim), jnp.float32),
            pltpu.SemaphoreType.DMA((_NBUF,)),
            pltpu.SemaphoreType.DMA((_NBUF,)),
        ],
    )
    def k(emb_hbm, out_hbm, bufs, gsem, ssem):
        wid = lax.axis_index("s") * nc + lax.axis_index("c")
        base = wid * rows_per_w

        def gather(i, b):
            return pltpu.async_copy(
                emb_hbm.at[pl.ds(base + i * _CHUNK, _CHUNK)],
                bufs.at[b],
                gsem.at[b],
            )

        def scatter(i, b):
            return pltpu.async_copy(
                bufs.at[b],
                out_hbm.at[pl.ds(base + i * _CHUNK, _CHUNK)],
                ssem.at[b],
            )

        g = [None] * nchunks
        s = [None] * nchunks
        for b in range(min(_NBUF, nchunks)):
            g[b] = gather(b, b)
        for i in range(nchunks):
            b = i % _NBUF
            g[i].wait()
            s[i] = scatter(i, b)
            nxt = i + _NBUF
            if nxt < nchunks:
                s[i].wait()
                g[nxt] = gather(nxt, b)
        for i in range(max(0, nchunks - _NBUF), nchunks):
            if s[i] is not None and i + _NBUF >= nchunks:
                s[i].wait()

    return k


def kernel(x, emb_weight):
    rows = x.shape[1]
    return _make_copy_kernel(rows, emb_weight.shape[1])(emb_weight)
